# baseline (speedup 1.0000x reference)
import functools

import jax
import jax.numpy as jnp
from jax import lax
from jax.experimental import pallas as pl
from jax.experimental.pallas import tpu as pltpu
from jax.experimental.pallas import tpu_sc as plsc


def _make_gather(B, V, D):
  info = plsc.get_sparse_core_info()
  NC, NS, L = info.num_cores, info.num_subcores, info.num_lanes
  NW = NC * NS
  assert B % (NW * L) == 0
  b_per_w = B // NW
  mesh = plsc.VectorSubcoreMesh(core_axis_name="c", subcore_axis_name="s")

  @functools.partial(
      pl.kernel,
      mesh=mesh,
      out_type=jax.ShapeDtypeStruct((B, D), jnp.float32),
      scratch_types=[
          pltpu.VMEM((b_per_w,), jnp.int32),
          pltpu.SemaphoreType.DMA,
      ],
  )
  def gather_kernel(idx_hbm, table_hbm, out_hbm, idx_v, sem):
    wid = lax.axis_index("s") * NC + lax.axis_index("c")
    base = wid * b_per_w
    pltpu.sync_copy(idx_hbm.at[pl.ds(base, b_per_w)], idx_v)

    def do_group(g, carry):
      vec = idx_v[pl.ds(g * L, L)]
      for j in range(L):
        row = lax.index_in_dim(vec, j, keepdims=False)
        pltpu.async_copy(
            table_hbm.at[pl.ds(row, 1)],
            out_hbm.at[pl.ds(base + g * L + j, 1)],
            sem,
        )
      return carry

    lax.fori_loop(0, b_per_w // L, do_group, 0)
    pltpu.make_async_copy(
        table_hbm.at[pl.ds(0, b_per_w)],
        out_hbm.at[pl.ds(base, b_per_w)],
        sem,
    ).wait()

  return gather_kernel


def kernel(indices, table):
  B, = indices.shape
  V, D = table.shape
  return _make_gather(B, V, D)(indices, table)


# speedup vs baseline: 1.0413x; 1.0413x over previous
import functools

import jax
import jax.numpy as jnp
from jax import lax
from jax.experimental import pallas as pl
from jax.experimental.pallas import tpu as pltpu
from jax.experimental.pallas import tpu_sc as plsc

_ISUB = 128


def _make_gather(B, V, D):
  info = plsc.get_sparse_core_info()
  NC, NS, L = info.num_cores, info.num_subcores, info.num_lanes
  NW = NC * NS
  assert B % (NW * L) == 0 and D == 2 * L and V % 4 == 0
  b_per_w = B // NW
  n_sub = b_per_w // _ISUB
  mesh = plsc.VectorSubcoreMesh(core_axis_name="c", subcore_axis_name="s")

  @functools.partial(
      pl.kernel,
      mesh=mesh,
      out_type=jax.ShapeDtypeStruct((B, D), jnp.float32),
      scratch_types=[
          pltpu.VMEM((b_per_w,), jnp.int32),
          pltpu.VMEM((b_per_w,), jnp.int32),
          pltpu.VMEM((b_per_w // 2, 4 * D), jnp.float32),
          pltpu.VMEM((b_per_w, D), jnp.float32),
          pltpu.SemaphoreType.DMA,
      ],
      compiler_params=pltpu.CompilerParams(needs_layout_passes=False),
  )
  def gather_kernel(idx_hbm, table4_hbm, out_hbm, idx_v, gidx_v, rows4_v,
                    stg_v, sem):
    s = lax.axis_index("s")
    c = lax.axis_index("c")
    wid = s * NC + c
    obase = wid * b_per_w
    pltpu.sync_copy(idx_hbm.at[pl.ds(obase, b_per_w)], idx_v)

    def lgroup(g, carry):
      iv = idx_v[pl.ds(g * L, L)]
      gidx_v[pl.ds(g * L, L)] = jnp.right_shift(iv, 2)
      return carry

    lax.fori_loop(0, b_per_w // L, lgroup, 0)

    lane = jnp.arange(0, L, dtype=jnp.int32)
    half = b_per_w // 2
    for h in range(2):
      copies = []
      for j in range(half // _ISUB):
        copies.append(
            pltpu.async_copy(
                table4_hbm.at[gidx_v.at[pl.ds(h * half + j * _ISUB, _ISUB)]],
                rows4_v.at[pl.ds(j * _ISUB, _ISUB)],
                sem,
            )
        )
      for cp in copies:
        cp.wait()

      def egroup(g, carry):
        jl = g * L + lane
        iv = idx_v[pl.ds(h * half + g * L, L)]
        colbase = jnp.bitwise_and(iv, 3) * D
        for cc in range(D):
          vals = plsc.load_gather(rows4_v, [jl, colbase + cc])
          plsc.store_scatter(
              stg_v, [h * half + jl, jnp.full((L,), cc, jnp.int32)], vals
          )
        return carry

      lax.fori_loop(0, half // L, egroup, 0)
    pltpu.sync_copy(stg_v, out_hbm.at[pl.ds(obase, b_per_w)])

  return gather_kernel


def kernel(indices, table):
  B, = indices.shape
  V, D = table.shape
  table4 = table.reshape(V // 4, 4 * D)
  return _make_gather(B, V, D)(indices, table4)


# speedup vs baseline: 1.0782x; 1.0354x over previous
import functools

import jax
import jax.numpy as jnp
from jax import lax
from jax.experimental import pallas as pl
from jax.experimental.pallas import tpu as pltpu
from jax.experimental.pallas import tpu_sc as plsc

_CHUNK = 128


def _make_gather(B, V, D):
  info = plsc.get_sparse_core_info()
  NC, NS = info.num_cores, info.num_subcores
  NW = NC * NS
  assert B % NW == 0
  b_per_w = B // NW
  n_chunks = b_per_w // _CHUNK
  assert b_per_w % _CHUNK == 0
  mesh = plsc.VectorSubcoreMesh(core_axis_name="c", subcore_axis_name="s")

  @functools.partial(
      pl.kernel,
      mesh=mesh,
      out_type=jax.ShapeDtypeStruct((B, D), jnp.float32),
      scratch_types=[
          pltpu.VMEM((b_per_w,), jnp.int32),
          pltpu.VMEM((b_per_w, D), jnp.float32),
          pltpu.SemaphoreType.DMA,
      ],
      compiler_params=pltpu.CompilerParams(use_tc_tiling_on_sc=False),
  )
  def gather_kernel(idx_hbm, table_hbm, out_hbm, idx_v, rows_v, sem):
    wid = lax.axis_index("s") * NC + lax.axis_index("c")
    base = wid * b_per_w
    pltpu.sync_copy(idx_hbm.at[pl.ds(base, b_per_w)], idx_v)
    copies = []
    for j in range(n_chunks):
      copies.append(
          pltpu.async_copy(
              table_hbm.at[idx_v.at[pl.ds(j * _CHUNK, _CHUNK)]],
              rows_v.at[pl.ds(j * _CHUNK, _CHUNK)],
              sem,
          )
      )
    for c in copies:
      c.wait()
    pltpu.sync_copy(rows_v, out_hbm.at[pl.ds(base, b_per_w)])

  return gather_kernel


def kernel(indices, table):
  B, = indices.shape
  V, D = table.shape
  return _make_gather(B, V, D)(indices, table)
